# baseline (speedup 1.0000x reference)
import functools
import math

import jax
import jax.numpy as jnp
from jax import lax
from jax.experimental import pallas as pl
from jax.experimental.pallas import tpu as pltpu
from jax.experimental.pallas import tpu_sc as plsc

L = 16


def _skey(b):
    return b ^ (lax.shift_right_arithmetic(b, 31) & jnp.int32(0x7FFFFFFF))


def _unskey(sk):
    return lax.bitcast_convert_type(
        sk ^ (lax.shift_right_arithmetic(sk, 31) & jnp.int32(0x7FFFFFFF)),
        jnp.float32)


def _make_kernel(B, N, K):
    info = plsc.get_sparse_core_info()
    NC, NS = info.num_cores, info.num_subcores
    NW = NC * NS
    assert B % NW == 0
    rows_per_w = B // NW
    mesh = plsc.VectorSubcoreMesh(core_axis_name="c", subcore_axis_name="s")

    def body(x_hbm, out_hbm, rowbuf, outbuf, listbuf, hist):
        wid = lax.axis_index("s") * NC + lax.axis_index("c")
        ones = jnp.ones((L,), jnp.int32)
        iota = lax.iota(jnp.int32, L)

        def zero_hist():
            @plsc.parallel_loop(0, 256, step=L, unroll=4)
            def _(j):
                hist[pl.ds(j, L)] = jnp.zeros((L,), jnp.int32)

        def scan_hist(kk):
            def scan_body(jj, carry):
                dstar, kprime, tail = carry
                j = 15 - jj
                h = hist[pl.ds(j * L, L)]
                c = plsc.cumsum(h)
                tot = jnp.sum(h)
                above = tail + tot - c
                cond = (above < kk) & (above + h >= kk)
                dsel = jnp.where(cond, iota + j * L, -1)
                ksel = jnp.where(cond, kk - above, -1)
                return (jnp.maximum(dstar, jnp.max(dsel)),
                        jnp.maximum(kprime, jnp.max(ksel)),
                        tail + tot)

            dstar, kprime, _ = lax.fori_loop(
                0, 256 // L, scan_body,
                (jnp.int32(-1), jnp.int32(-1), jnp.int32(0)))
            return dstar, kprime

        def do_row(i, _):
            row = wid * rows_per_w + i
            pltpu.sync_copy(x_hbm.at[row], rowbuf)

            zero_hist()

            @plsc.parallel_loop(0, N, step=L, unroll=8)
            def _(j):
                sk = _skey(lax.bitcast_convert_type(rowbuf[pl.ds(j, L)],
                                                    jnp.int32))
                bucket = lax.shift_right_arithmetic(sk, 24) + 128
                plsc.addupdate_scatter(hist, [bucket], ones)

            d1, k1 = scan_hist(jnp.int32(K))
            t1 = d1 - 128

            carry0 = (jnp.zeros((L,), jnp.int32), jnp.zeros((L,), jnp.float32))

            @plsc.parallel_loop(0, N, step=L, unroll=8, carry=carry0)
            def compact_out(j, carry):
                off, vacc = carry
                v = rowbuf[pl.ds(j, L)]
                sk = _skey(lax.bitcast_convert_type(v, jnp.int32))
                top = lax.shift_right_arithmetic(sk, 24)
                m = top == t1
                mi = m.astype(jnp.int32)
                pos = off + plsc.cumsum(mi) - mi
                plsc.store_scatter(listbuf, [pos], sk, mask=m)
                off = off + plsc.all_reduce_population_count(m)
                vacc = vacc + jnp.where(top > t1, v, jnp.float32(0))
                return off, vacc

            off, vacc = compact_out
            m_cnt = jnp.max(off)
            sum_above = jnp.sum(vacc)

            kk = k1
            t = t1
            for lvl in range(1, 4):
                shift = 24 - 8 * lvl
                zero_hist()

                @plsc.parallel_loop(0, m_cnt, step=L, unroll=2)
                def _(j, lvl=lvl, shift=shift, t=t):
                    sk = listbuf[pl.ds(j, L)]
                    valid = iota + j < m_cnt
                    if lvl > 1:
                        prefix = lax.shift_right_arithmetic(sk, shift + 8)
                        valid = valid & (prefix == t)
                    bucket = lax.shift_right_arithmetic(sk, shift) & 255
                    plsc.addupdate_scatter(hist, [bucket], ones, mask=valid)

                dstar, kk = scan_hist(kk)
                t = (t << 8) | dstar

            thrv = _unskey(jnp.full((L,), t, jnp.int32))

            fin0 = (jnp.zeros((L,), jnp.float32), jnp.zeros((L,), jnp.int32))

            @plsc.parallel_loop(0, m_cnt, step=L, unroll=2, carry=fin0)
            def fin(j, carry):
                sacc, ecnt = carry
                sk = listbuf[pl.ds(j, L)]
                valid = iota + j < m_cnt
                gt = valid & (sk > t)
                eq = valid & (sk == t)
                sacc = sacc + jnp.where(gt, _unskey(sk), jnp.float32(0))
                ecnt = ecnt + jnp.where(eq, 1, 0)
                return sacc, ecnt

            sacc, ecnt = fin
            total = (sum_above + jnp.sum(sacc)
                     + jnp.sum(ecnt).astype(jnp.float32) * thrv[0])
            scale = jnp.full((L,), jnp.float32(N)) / jnp.full((L,), total)

            @plsc.parallel_loop(0, N, step=L, unroll=8)
            def _(j):
                v = rowbuf[pl.ds(j, L)]
                outbuf[pl.ds(j, L)] = jnp.where(v >= thrv, v * scale,
                                                jnp.float32(0))

            pltpu.sync_copy(outbuf, out_hbm.at[row])
            return 0

        lax.fori_loop(0, rows_per_w, do_row, 0)

    return pl.kernel(
        body,
        out_type=jax.ShapeDtypeStruct((B, N), jnp.float32),
        mesh=mesh,
        compiler_params=pltpu.CompilerParams(needs_layout_passes=False),
        scratch_types=[
            pltpu.VMEM((N,), jnp.float32),
            pltpu.VMEM((N,), jnp.float32),
            pltpu.VMEM((N,), jnp.int32),
            pltpu.VMEM((256,), jnp.int32),
        ],
    )


@jax.jit
def kernel(x):
    B, N = x.shape
    K = int(math.ceil(0.1 * N))
    return _make_kernel(B, N, K)(x)


# speedup vs baseline: 1.0311x; 1.0311x over previous
import functools
import math

import jax
import jax.numpy as jnp
from jax import lax
from jax.experimental import pallas as pl
from jax.experimental.pallas import tpu as pltpu
from jax.experimental.pallas import tpu_sc as plsc

L = 16


def _skey(b):
    return b ^ (lax.shift_right_arithmetic(b, 31) & jnp.int32(0x7FFFFFFF))


def _unskey(sk):
    return lax.bitcast_convert_type(
        sk ^ (lax.shift_right_arithmetic(sk, 31) & jnp.int32(0x7FFFFFFF)),
        jnp.float32)


def _make_kernel(B, N, K):
    info = plsc.get_sparse_core_info()
    NC, NS = info.num_cores, info.num_subcores
    NW = NC * NS
    assert B % NW == 0
    rows_per_w = B // NW
    mesh = plsc.VectorSubcoreMesh(core_axis_name="c", subcore_axis_name="s")

    def body(x_hbm, out_hbm, rowbuf, outbuf, listbuf, hist):
        wid = lax.axis_index("s") * NC + lax.axis_index("c")
        ones = jnp.ones((L,), jnp.int32)
        iota = lax.iota(jnp.int32, L)

        def zero_hist():
            @plsc.parallel_loop(0, 256, step=L, unroll=4)
            def _(j):
                hist[pl.ds(j, L)] = jnp.zeros((L,), jnp.int32)

        def scan_hist(kk):
            def scan_body(jj, carry):
                dstar, kprime, tail = carry
                j = 15 - jj
                h = hist[pl.ds(j * L, L)]
                c = plsc.cumsum(h)
                tot = jnp.sum(h)
                above = tail + tot - c
                cond = (above < kk) & (above + h >= kk)
                dsel = jnp.where(cond, iota + j * L, -1)
                ksel = jnp.where(cond, kk - above, -1)
                return (jnp.maximum(dstar, jnp.max(dsel)),
                        jnp.maximum(kprime, jnp.max(ksel)),
                        tail + tot)

            dstar, kprime, _ = lax.fori_loop(
                0, 256 // L, scan_body,
                (jnp.int32(-1), jnp.int32(-1), jnp.int32(0)))
            return dstar, kprime

        def do_row(i, _):
            row = wid * rows_per_w + i
            pltpu.sync_copy(x_hbm.at[row], rowbuf)

            zero_hist()

            @plsc.parallel_loop(0, N, step=L, unroll=8)
            def _(j):
                sk = _skey(lax.bitcast_convert_type(rowbuf[pl.ds(j, L)],
                                                    jnp.int32))
                bucket = lax.shift_right_arithmetic(sk, 24) + 128
                plsc.addupdate_scatter(hist, [bucket], ones)

            d1, k1 = scan_hist(jnp.int32(K))
            t1 = d1 - 128

            carry0 = (jnp.int32(0), jnp.zeros((L,), jnp.float32))

            @plsc.parallel_loop(0, N, step=L, unroll=8, carry=carry0)
            def compact_out(j, carry):
                off, vacc = carry
                v = rowbuf[pl.ds(j, L)]
                sk = _skey(lax.bitcast_convert_type(v, jnp.int32))
                top = lax.shift_right_arithmetic(sk, 24)
                m = top == t1
                plsc.store_compressed(listbuf.at[pl.ds(off, L)], sk, mask=m)
                off = off + plsc.all_reduce_population_count(m)[0]
                vacc = vacc + jnp.where(top > t1, v, jnp.float32(0))
                return off, vacc

            m_cnt, vacc = compact_out
            sum_above = jnp.sum(vacc)

            kk = k1
            t = t1
            for lvl in range(1, 4):
                shift = 24 - 8 * lvl
                zero_hist()

                @plsc.parallel_loop(0, m_cnt, step=L, unroll=2)
                def _(j, lvl=lvl, shift=shift, t=t):
                    sk = listbuf[pl.ds(j, L)]
                    valid = iota + j < m_cnt
                    if lvl > 1:
                        prefix = lax.shift_right_arithmetic(sk, shift + 8)
                        valid = valid & (prefix == t)
                    bucket = lax.shift_right_arithmetic(sk, shift) & 255
                    plsc.addupdate_scatter(hist, [bucket], ones, mask=valid)

                dstar, kk = scan_hist(kk)
                t = (t << 8) | dstar

            thrv = _unskey(jnp.full((L,), t, jnp.int32))

            fin0 = (jnp.zeros((L,), jnp.float32), jnp.zeros((L,), jnp.int32))

            @plsc.parallel_loop(0, m_cnt, step=L, unroll=2, carry=fin0)
            def fin(j, carry):
                sacc, ecnt = carry
                sk = listbuf[pl.ds(j, L)]
                valid = iota + j < m_cnt
                gt = valid & (sk > t)
                eq = valid & (sk == t)
                sacc = sacc + jnp.where(gt, _unskey(sk), jnp.float32(0))
                ecnt = ecnt + jnp.where(eq, 1, 0)
                return sacc, ecnt

            sacc, ecnt = fin
            total = (sum_above + jnp.sum(sacc)
                     + jnp.sum(ecnt).astype(jnp.float32) * thrv[0])
            scale = jnp.full((L,), jnp.float32(N)) / jnp.full((L,), total)

            @plsc.parallel_loop(0, N, step=L, unroll=8)
            def _(j):
                v = rowbuf[pl.ds(j, L)]
                outbuf[pl.ds(j, L)] = jnp.where(v >= thrv, v * scale,
                                                jnp.float32(0))

            pltpu.sync_copy(outbuf, out_hbm.at[row])
            return 0

        lax.fori_loop(0, rows_per_w, do_row, 0)

    return pl.kernel(
        body,
        out_type=jax.ShapeDtypeStruct((B, N), jnp.float32),
        mesh=mesh,
        compiler_params=pltpu.CompilerParams(needs_layout_passes=False),
        scratch_types=[
            pltpu.VMEM((N,), jnp.float32),
            pltpu.VMEM((N,), jnp.float32),
            pltpu.VMEM((N,), jnp.int32),
            pltpu.VMEM((256,), jnp.int32),
        ],
    )


@jax.jit
def kernel(x):
    B, N = x.shape
    K = int(math.ceil(0.1 * N))
    return _make_kernel(B, N, K)(x)


# speedup vs baseline: 1.1487x; 1.1140x over previous
import functools
import math

import jax
import jax.numpy as jnp
from jax import lax
from jax.experimental import pallas as pl
from jax.experimental.pallas import tpu as pltpu
from jax.experimental.pallas import tpu_sc as plsc

L = 16


def _skey(b):
    return b ^ (lax.shift_right_arithmetic(b, 31) & jnp.int32(0x7FFFFFFF))


def _unskey(sk):
    return lax.bitcast_convert_type(
        sk ^ (lax.shift_right_arithmetic(sk, 31) & jnp.int32(0x7FFFFFFF)),
        jnp.float32)


def _make_kernel(B, N, K):
    info = plsc.get_sparse_core_info()
    NC, NS = info.num_cores, info.num_subcores
    NW = NC * NS
    assert B % NW == 0
    rows_per_w = B // NW
    CAP = N - 1024
    mesh = plsc.VectorSubcoreMesh(core_axis_name="c", subcore_axis_name="s")

    def body(x_hbm, out_hbm, buf0, buf1, buf2, listbuf, hist,
             sem0, sem1, sem2):
        wid = lax.axis_index("s") * NC + lax.axis_index("c")
        bufs = (buf0, buf1, buf2)
        sems = (sem0, sem1, sem2)
        ones = jnp.ones((L,), jnp.int32)
        iota = lax.iota(jnp.int32, L)

        def zero_hist():
            @plsc.parallel_loop(0, 256, step=L, unroll=4)
            def _(j):
                hist[pl.ds(j, L)] = jnp.zeros((L,), jnp.int32)

        def scan_hist(kk):
            def scan_body(jj, carry):
                dvec, kvec, tail = carry
                j = 15 - jj
                h = hist[pl.ds(j * L, L)]
                c = plsc.cumsum(h)
                tot = c[L - 1]
                above = tail + tot - c
                cond = (above < kk) & (above + h >= kk)
                dvec = jnp.maximum(dvec, jnp.where(cond, iota + j * L, -1))
                kvec = jnp.maximum(kvec, jnp.where(cond, kk - above, -1))
                return dvec, kvec, tail + tot

            dvec, kvec, _ = lax.fori_loop(
                0, 256 // L, scan_body,
                (jnp.full((L,), -1, jnp.int32), jnp.full((L,), -1, jnp.int32),
                 jnp.int32(0)))
            return jnp.max(dvec), jnp.max(kvec)

        def process(rowbuf):
            zero_hist()

            @plsc.parallel_loop(0, N, step=L, unroll=8)
            def _(j):
                sk = _skey(lax.bitcast_convert_type(rowbuf[pl.ds(j, L)],
                                                    jnp.int32))
                bucket = lax.shift_right_arithmetic(sk, 24) + 128
                plsc.addupdate_scatter(hist, [bucket], ones)

            d1, k1 = scan_hist(jnp.int32(K))
            t1 = d1 - 128

            carry0 = (jnp.int32(0), jnp.zeros((L,), jnp.float32))

            @plsc.parallel_loop(0, N, step=L, unroll=8, carry=carry0)
            def compact_out(j, carry):
                off, vacc = carry
                v = rowbuf[pl.ds(j, L)]
                sk = _skey(lax.bitcast_convert_type(v, jnp.int32))
                top = lax.shift_right_arithmetic(sk, 24)
                m = top == t1
                plsc.store_compressed(
                    listbuf.at[pl.ds(jnp.minimum(off, CAP - L), L)], sk,
                    mask=m)
                off = off + plsc.all_reduce_population_count(m)[0]
                vacc = vacc + jnp.where(top > t1, v, jnp.float32(0))
                return off, vacc

            m_cnt, vacc = compact_out
            m_cnt = jnp.minimum(m_cnt, CAP)
            sum1 = jnp.sum(vacc)

            zero_hist()

            @plsc.parallel_loop(0, m_cnt, step=L, unroll=4)
            def _(j):
                sk = listbuf[pl.ds(j, L)]
                valid = iota + j < m_cnt
                bucket = lax.shift_right_arithmetic(sk, 16) & 255
                plsc.addupdate_scatter(hist, [bucket], ones, mask=valid)

            d2, k2 = scan_hist(k1)
            t2 = (t1 << 8) | d2

            carry2 = (jnp.int32(0), jnp.zeros((L,), jnp.float32))

            @plsc.parallel_loop(0, m_cnt, step=L, unroll=4, carry=carry2)
            def compact2_out(j, carry):
                off, vacc2 = carry
                sk = listbuf[pl.ds(j, L)]
                valid = iota + j < m_cnt
                pre = lax.shift_right_arithmetic(sk, 16)
                m = valid & (pre == t2)
                plsc.store_compressed(
                    listbuf.at[pl.ds(jnp.minimum(off, CAP - L), L)], sk,
                    mask=m)
                off = off + plsc.all_reduce_population_count(m)[0]
                vacc2 = vacc2 + jnp.where(valid & (pre > t2), _unskey(sk),
                                          jnp.float32(0))
                return off, vacc2

            m2, vacc2 = compact2_out
            m2 = jnp.minimum(m2, CAP)
            sum2 = jnp.sum(vacc2)

            zero_hist()

            @plsc.parallel_loop(0, m2, step=L, unroll=2)
            def _(j):
                sk = listbuf[pl.ds(j, L)]
                valid = iota + j < m2
                bucket = lax.shift_right_arithmetic(sk, 8) & 255
                plsc.addupdate_scatter(hist, [bucket], ones, mask=valid)

            d3, k3 = scan_hist(k2)
            t3 = (t2 << 8) | d3
            zero_hist()

            @plsc.parallel_loop(0, m2, step=L, unroll=2)
            def _(j):
                sk = listbuf[pl.ds(j, L)]
                valid = ((iota + j < m2)
                         & (lax.shift_right_arithmetic(sk, 8) == t3))
                plsc.addupdate_scatter(hist, [sk & 255], ones, mask=valid)

            d4, _k4 = scan_hist(k3)
            t = (t3 << 8) | d4
            thrv = _unskey(jnp.full((L,), t, jnp.int32))

            fin0 = (jnp.zeros((L,), jnp.float32), jnp.zeros((L,), jnp.int32))

            @plsc.parallel_loop(0, m2, step=L, unroll=2, carry=fin0)
            def fin(j, carry):
                sacc, ecnt = carry
                sk = listbuf[pl.ds(j, L)]
                valid = iota + j < m2
                sacc = sacc + jnp.where(valid & (sk > t), _unskey(sk),
                                        jnp.float32(0))
                ecnt = ecnt + jnp.where(valid & (sk == t), 1, 0)
                return sacc, ecnt

            sacc, ecnt = fin
            total = (sum1 + sum2 + jnp.sum(sacc)
                     + jnp.sum(ecnt).astype(jnp.float32) * thrv[0])
            scale = jnp.full((L,), jnp.float32(N)) / jnp.full((L,), total)

            @plsc.parallel_loop(0, N, step=L, unroll=8)
            def _(j):
                v = rowbuf[pl.ds(j, L)]
                rowbuf[pl.ds(j, L)] = jnp.where(v >= thrv, v * scale,
                                                jnp.float32(0))

        first = wid * rows_per_w
        pending = [None, None, None]
        cin = [None] * rows_per_w
        cin[0] = pltpu.async_copy(x_hbm.at[first], bufs[0], sems[0])
        for i in range(rows_per_w):
            p = i % 3
            if i + 1 < rows_per_w:
                q = (i + 1) % 3
                if pending[q] is not None:
                    pending[q].wait()
                    pending[q] = None
                cin[i + 1] = pltpu.async_copy(x_hbm.at[first + i + 1],
                                              bufs[q], sems[q])
            cin[i].wait()
            process(bufs[p])
            pending[p] = pltpu.async_copy(bufs[p], out_hbm.at[first + i],
                                          sems[p])
        for h in pending:
            if h is not None:
                h.wait()

    return pl.kernel(
        body,
        out_type=jax.ShapeDtypeStruct((B, N), jnp.float32),
        mesh=mesh,
        compiler_params=pltpu.CompilerParams(needs_layout_passes=False),
        scratch_types=[
            pltpu.VMEM((N,), jnp.float32),
            pltpu.VMEM((N,), jnp.float32),
            pltpu.VMEM((N,), jnp.float32),
            pltpu.VMEM((CAP,), jnp.int32),
            pltpu.VMEM((256,), jnp.int32),
            pltpu.SemaphoreType.DMA,
            pltpu.SemaphoreType.DMA,
            pltpu.SemaphoreType.DMA,
        ],
    )


@jax.jit
def kernel(x):
    B, N = x.shape
    K = int(math.ceil(0.1 * N))
    return _make_kernel(B, N, K)(x)


# speedup vs baseline: 1.2151x; 1.0578x over previous
import functools
import math

import jax
import jax.numpy as jnp
from jax import lax
from jax.experimental import pallas as pl
from jax.experimental.pallas import tpu as pltpu
from jax.experimental.pallas import tpu_sc as plsc

L = 16


def _skey(b):
    return b ^ (lax.shift_right_arithmetic(b, 31) & jnp.int32(0x7FFFFFFF))


def _unskey(sk):
    return lax.bitcast_convert_type(
        sk ^ (lax.shift_right_arithmetic(sk, 31) & jnp.int32(0x7FFFFFFF)),
        jnp.float32)


def _make_kernel(B, N, K):
    info = plsc.get_sparse_core_info()
    NC, NS = info.num_cores, info.num_subcores
    NW = NC * NS
    assert B % NW == 0
    rows_per_w = B // NW
    CAP = N - 1024
    mesh = plsc.VectorSubcoreMesh(core_axis_name="c", subcore_axis_name="s")

    def body(x_hbm, out_hbm, buf0, buf1, buf2, listbuf, hist,
             sem0, sem1, sem2):
        wid = lax.axis_index("s") * NC + lax.axis_index("c")
        bufs = (buf0, buf1, buf2)
        sems = (sem0, sem1, sem2)
        ones = jnp.ones((L,), jnp.int32)
        iota = lax.iota(jnp.int32, L)

        def zero_hist():
            @plsc.parallel_loop(0, 256, step=L, unroll=4)
            def _(j):
                hist[pl.ds(j, L)] = jnp.zeros((L,), jnp.int32)

        def scan_hist(kk):
            def scan_body(jj, carry):
                dvec, kvec, tail = carry
                j = 15 - jj
                h = hist[pl.ds(j * L, L)]
                c = plsc.cumsum(h)
                tot = c[L - 1]
                above = tail + tot - c
                cond = (above < kk) & (above + h >= kk)
                dvec = jnp.maximum(dvec, jnp.where(cond, iota + j * L, -1))
                kvec = jnp.maximum(kvec, jnp.where(cond, kk - above, -1))
                return dvec, kvec, tail + tot

            dvec, kvec, _ = lax.fori_loop(
                0, 256 // L, scan_body,
                (jnp.full((L,), -1, jnp.int32), jnp.full((L,), -1, jnp.int32),
                 jnp.int32(0)))
            return jnp.max(dvec), jnp.max(kvec)

        def process(rowbuf):
            zero_hist()

            @plsc.parallel_loop(0, N, step=L, unroll=8)
            def _(j):
                sk = _skey(lax.bitcast_convert_type(rowbuf[pl.ds(j, L)],
                                                    jnp.int32))
                bucket = lax.shift_right_arithmetic(sk, 24) + 128
                plsc.addupdate_scatter(hist, [bucket], ones)

            d1, k1 = scan_hist(jnp.int32(K))
            t1 = d1 - 128

            zero_hist()
            carry0 = (jnp.int32(0), jnp.zeros((L,), jnp.float32))

            @plsc.parallel_loop(0, N, step=L, unroll=8, carry=carry0)
            def compact_out(j, carry):
                off, vacc = carry
                v = rowbuf[pl.ds(j, L)]
                sk = _skey(lax.bitcast_convert_type(v, jnp.int32))
                top = lax.shift_right_arithmetic(sk, 24)
                m = top == t1
                plsc.store_compressed(
                    listbuf.at[pl.ds(jnp.minimum(off, CAP - L), L)], sk,
                    mask=m)
                bucket = lax.shift_right_arithmetic(sk, 16) & 255
                plsc.addupdate_scatter(hist, [bucket], ones, mask=m)
                off = off + plsc.all_reduce_population_count(m)[0]
                vacc = vacc + jnp.where(top > t1, v, jnp.float32(0))
                return off, vacc

            m_cnt, vacc = compact_out
            m_cnt = jnp.minimum(m_cnt, CAP)
            sum1 = jnp.sum(vacc)

            d2, k2 = scan_hist(k1)
            t2 = (t1 << 8) | d2

            zero_hist()
            carry2 = (jnp.int32(0), jnp.zeros((L,), jnp.float32))

            @plsc.parallel_loop(0, m_cnt, step=L, unroll=4, carry=carry2)
            def compact2_out(j, carry):
                off, vacc2 = carry
                sk = listbuf[pl.ds(j, L)]
                valid = iota + j < m_cnt
                pre = lax.shift_right_arithmetic(sk, 16)
                m = valid & (pre == t2)
                plsc.store_compressed(
                    listbuf.at[pl.ds(jnp.minimum(off, CAP - L), L)], sk,
                    mask=m)
                bucket = lax.shift_right_arithmetic(sk, 8) & 255
                plsc.addupdate_scatter(hist, [bucket], ones, mask=m)
                off = off + plsc.all_reduce_population_count(m)[0]
                vacc2 = vacc2 + jnp.where(valid & (pre > t2), _unskey(sk),
                                          jnp.float32(0))
                return off, vacc2

            m2, vacc2 = compact2_out
            m2 = jnp.minimum(m2, CAP)
            sum2 = jnp.sum(vacc2)

            d3, k3 = scan_hist(k2)
            t3 = (t2 << 8) | d3
            zero_hist()

            @plsc.parallel_loop(0, m2, step=L, unroll=2)
            def _(j):
                sk = listbuf[pl.ds(j, L)]
                valid = ((iota + j < m2)
                         & (lax.shift_right_arithmetic(sk, 8) == t3))
                plsc.addupdate_scatter(hist, [sk & 255], ones, mask=valid)

            d4, _k4 = scan_hist(k3)
            t = (t3 << 8) | d4
            thrv = _unskey(jnp.full((L,), t, jnp.int32))

            fin0 = (jnp.zeros((L,), jnp.float32), jnp.zeros((L,), jnp.int32))

            @plsc.parallel_loop(0, m2, step=L, unroll=2, carry=fin0)
            def fin(j, carry):
                sacc, ecnt = carry
                sk = listbuf[pl.ds(j, L)]
                valid = iota + j < m2
                sacc = sacc + jnp.where(valid & (sk > t), _unskey(sk),
                                        jnp.float32(0))
                ecnt = ecnt + jnp.where(valid & (sk == t), 1, 0)
                return sacc, ecnt

            sacc, ecnt = fin
            total = (sum1 + sum2 + jnp.sum(sacc)
                     + jnp.sum(ecnt).astype(jnp.float32) * thrv[0])
            scale = jnp.full((L,), jnp.float32(N)) / jnp.full((L,), total)

            @plsc.parallel_loop(0, N, step=L, unroll=8)
            def _(j):
                v = rowbuf[pl.ds(j, L)]
                rowbuf[pl.ds(j, L)] = jnp.where(v >= thrv, v * scale,
                                                jnp.float32(0))

        first = wid * rows_per_w
        pending = [None, None, None]
        cin = [None] * rows_per_w
        cin[0] = pltpu.async_copy(x_hbm.at[first], bufs[0], sems[0])
        for i in range(rows_per_w):
            p = i % 3
            if i + 1 < rows_per_w:
                q = (i + 1) % 3
                if pending[q] is not None:
                    pending[q].wait()
                    pending[q] = None
                cin[i + 1] = pltpu.async_copy(x_hbm.at[first + i + 1],
                                              bufs[q], sems[q])
            cin[i].wait()
            process(bufs[p])
            pending[p] = pltpu.async_copy(bufs[p], out_hbm.at[first + i],
                                          sems[p])
        for h in pending:
            if h is not None:
                h.wait()

    return pl.kernel(
        body,
        out_type=jax.ShapeDtypeStruct((B, N), jnp.float32),
        mesh=mesh,
        compiler_params=pltpu.CompilerParams(needs_layout_passes=False),
        scratch_types=[
            pltpu.VMEM((N,), jnp.float32),
            pltpu.VMEM((N,), jnp.float32),
            pltpu.VMEM((N,), jnp.float32),
            pltpu.VMEM((CAP,), jnp.int32),
            pltpu.VMEM((256,), jnp.int32),
            pltpu.SemaphoreType.DMA,
            pltpu.SemaphoreType.DMA,
            pltpu.SemaphoreType.DMA,
        ],
    )


@jax.jit
def kernel(x):
    B, N = x.shape
    K = int(math.ceil(0.1 * N))
    return _make_kernel(B, N, K)(x)
